# baseline (speedup 1.0000x reference)
import functools

import jax
import jax.numpy as jnp
from jax import lax
from jax.experimental import pallas as pl
from jax.experimental.pallas import tpu as pltpu
from jax.experimental.pallas import tpu_sc as plsc

N_NODES = 10000
N_PAD = 10000
N_EDGES = 320000
N_TILES = 16
EPT = N_EDGES // N_TILES
ROWS_PER_TILE = N_PAD // N_TILES

_CFG = {
    128: (192, 105, 3),
    64: (320, 63, 3),
}


def _mm_body(x_ref, w_ref, o_ref, *, relu_in):
    x = x_ref[...]
    if relu_in:
        x = jnp.maximum(x, 0.0)
    o_ref[...] = jnp.dot(x, w_ref[...], preferred_element_type=jnp.float32)


def _matmul(x, w, relu_in=False, bm=1000):
    n, k = x.shape
    f = w.shape[1]
    return pl.pallas_call(
        functools.partial(_mm_body, relu_in=relu_in),
        grid=(n // bm,),
        in_specs=[
            pl.BlockSpec((bm, k), lambda i: (i, 0)),
            pl.BlockSpec((k, f), lambda i: (0, 0)),
        ],
        out_specs=pl.BlockSpec((bm, f), lambda i: (i, 0)),
        out_shape=jax.ShapeDtypeStruct((n, f), jnp.float32),
    )(x, w)


def _norm_body(x_ref, o_ref):
    y = jnp.maximum(x_ref[...], 0.0)
    nrm = jnp.sqrt(jnp.sum(y * y, axis=1, keepdims=True))
    o_ref[...] = y / jnp.maximum(nrm, 1e-12)


def _relu_l2norm(x, bm=1000):
    n, f = x.shape
    return pl.pallas_call(
        _norm_body,
        grid=(n // bm,),
        in_specs=[pl.BlockSpec((bm, f), lambda i: (i, 0))],
        out_specs=pl.BlockSpec((bm, f), lambda i: (i, 0)),
        out_shape=jax.ShapeDtypeStruct((n, f), jnp.float32),
    )(x)


def _relu_body(x_ref, o_ref):
    o_ref[...] = jnp.maximum(x_ref[...], 0.0)


def _relu(x, bm=1000):
    n, f = x.shape
    return pl.pallas_call(
        _relu_body,
        grid=(n // bm,),
        in_specs=[pl.BlockSpec((bm, f), lambda i: (i, 0))],
        out_specs=pl.BlockSpec((bm, f), lambda i: (i, 0)),
        out_shape=jax.ShapeDtypeStruct((n, f), jnp.float32),
    )(x)


def _sc_spmm(sup_sr, sup_tg, e_sr, e_tg, feat):
    mesh = plsc.VectorSubcoreMesh(core_axis_name="c", subcore_axis_name="s")
    nvec = feat // 16
    b_edges, n_chunks, stage_rows = _CFG[feat]
    n_stages = n_chunks // stage_rows

    @functools.partial(
        pl.kernel,
        mesh=mesh,
        out_type=(
            jax.ShapeDtypeStruct((N_PAD, feat), jnp.float32),
            jax.ShapeDtypeStruct((N_PAD, feat), jnp.float32),
        ),
        scratch_types=[
            pltpu.VMEM((stage_rows, 3, b_edges), jnp.int32),
            pltpu.VMEM((b_edges, feat), jnp.float32),
            pltpu.VMEM((b_edges, feat), jnp.float32),
            pltpu.VMEM_SHARED((N_PAD, feat), jnp.float32),
            pltpu.SemaphoreType.DMA,
            pltpu.SemaphoreType.DMA,
            pltpu.SemaphoreType.DMA,
            pltpu.SemaphoreType.DMA,
        ],
        compiler_params=pltpu.CompilerParams(use_tc_tiling_on_sc=False),
    )
    def spmm(sup_sr_h, sup_tg_h, esr_h, etg_h, out_sr_h, out_tg_h,
             idx_v, rows_a, rows_b, acc, gsem_a, gsem_b, ssem_a, ssem_b):
        g = lax.axis_index("c")
        sid = lax.axis_index("s")
        bufs = (rows_a, rows_b)
        gsems = (gsem_a, gsem_b)
        ssems = (ssem_a, ssem_b)

        zr = min(b_edges, 125)
        def zrow(r, c):
            for j in range(nvec):
                rows_a[r, pl.ds(j * 16, 16)] = jnp.zeros((16,), jnp.float32)
            return c
        lax.fori_loop(0, zr, zrow, 0)
        for i in range(ROWS_PER_TILE // zr):
            pltpu.sync_copy(
                rows_a.at[pl.ds(0, zr)],
                acc.at[pl.ds(sid * ROWS_PER_TILE + i * zr, zr)])
        plsc.subcore_barrier()

        def run(sup_h, e_h, out_h):
            def stage(si, c):
                sl = pl.ds(si * stage_rows, stage_rows)
                pltpu.sync_copy(e_h.at[sid, sl], idx_v)

                gd = {0: pltpu.async_copy(sup_h.at[idx_v.at[0, 0]], bufs[0],
                                          gsems[0])}
                sd = {}
                for r in range(stage_rows):
                    b = r % 2
                    gd[r].wait()
                    if r + 1 < stage_rows:
                        nb = (r + 1) % 2
                        if r >= 1:
                            sd[r - 1].wait()
                        gd[r + 1] = pltpu.async_copy(
                            sup_h.at[idx_v.at[r + 1, 0]], bufs[nb], gsems[nb])

                    def edge_group(gi, c3, _r=r, _b=b):
                        wbits = idx_v[_r, 2, pl.ds(gi * 16, 16)]
                        wvec = jax.lax.bitcast_convert_type(wbits, jnp.float32)
                        for i in range(16):
                            wv = wvec[i]
                            e = gi * 16 + i
                            for j in range(nvec):
                                fl = pl.ds(j * 16, 16)
                                bufs[_b][e, fl] = bufs[_b][e, fl] * wv
                        return c3
                    lax.fori_loop(0, b_edges // 16, edge_group, 0)
                    sd[r] = pltpu.async_copy(bufs[b], acc.at[idx_v.at[r, 1]],
                                             ssems[b], add=True)
                if stage_rows >= 2:
                    sd[stage_rows - 2].wait()
                sd[stage_rows - 1].wait()
                return c
            lax.fori_loop(0, n_stages, stage, 0)
            plsc.subcore_barrier()
            pltpu.sync_copy(
                acc.at[pl.ds(sid * ROWS_PER_TILE, ROWS_PER_TILE)],
                out_h.at[pl.ds(sid * ROWS_PER_TILE, ROWS_PER_TILE)])

        @pl.when(g == 0)
        def _():
            run(sup_sr_h, esr_h, out_sr_h)

        @pl.when(g == 1)
        def _():
            run(sup_tg_h, etg_h, out_tg_h)

    return spmm(sup_sr, sup_tg, e_sr, e_tg)


def _prep(ei, ew, feat):
    b_edges, n_chunks, _ = _CFG[feat]
    ept_pad = n_chunks * b_edges

    def shape_one(a):
        a = a.reshape(N_TILES, EPT)
        if ept_pad > EPT:
            a = jnp.pad(a, ((0, 0), (0, ept_pad - EPT)))
        return a.reshape(N_TILES, n_chunks, b_edges)

    src = shape_one(jnp.asarray(ei[0], jnp.int32))
    dst = shape_one(jnp.asarray(ei[1], jnp.int32))
    w = shape_one(jax.lax.bitcast_convert_type(ew.astype(jnp.float32),
                                               jnp.int32))
    return jnp.stack([src, dst, w], axis=2)


def kernel(edge_index_sr, edge_index_tg, edge_weight_sr, edge_weight_tg,
           attr_weight_sr, attr_weight_tg, emb_sr, emb_tg,
           W_s0, W_s1, W_a11, W_a12, W_a2):
    esr128 = _prep(edge_index_sr, edge_weight_sr, 128)
    etg128 = _prep(edge_index_tg, edge_weight_tg, 128)
    esr64 = _prep(edge_index_sr, edge_weight_sr, 64)
    etg64 = _prep(edge_index_tg, edge_weight_tg, 64)

    s_sr = _matmul(emb_sr, W_s0)
    s_tg = _matmul(emb_tg, W_s0)
    a1_sr, a1_tg = _sc_spmm(s_sr, s_tg, esr128, etg128, 128)
    s2_sr = _matmul(a1_sr, W_s1, relu_in=True)
    s2_tg = _matmul(a1_tg, W_s1, relu_in=True)
    a2_sr, a2_tg = _sc_spmm(s2_sr, s2_tg, esr128, etg128, 128)
    sr_s = _relu_l2norm(a2_sr)
    tg_s = _relu_l2norm(a2_tg)

    t_sr = _matmul(attr_weight_sr, W_a11)
    t_tg = _matmul(attr_weight_tg, W_a12)
    b1_sr, b1_tg = _sc_spmm(t_sr, t_tg, esr64, etg64, 64)
    t2_sr = _matmul(b1_sr, W_a2, relu_in=True)
    t2_tg = _matmul(b1_tg, W_a2, relu_in=True)
    b2_sr, b2_tg = _sc_spmm(t2_sr, t2_tg, esr64, etg64, 64)
    sr_a = _relu(b2_sr)
    tg_a = _relu(b2_tg)

    return (sr_s, tg_s, sr_a, tg_a)


# speedup vs baseline: 1.1803x; 1.1803x over previous
import functools

import jax
import jax.numpy as jnp
from jax import lax
from jax.experimental import pallas as pl
from jax.experimental.pallas import tpu as pltpu
from jax.experimental.pallas import tpu_sc as plsc

N_NODES = 10000
N_PAD = 10240
N_EDGES = 320000
N_TILES = 16
EPT = N_EDGES // N_TILES
ROWS_PER_TILE = N_PAD // N_TILES

_CFG = {
    128: (160, 125, 5),
    64: (160, 125, 5),
}


def _mm_body(x_ref, w_ref, o_ref, *, relu_in):
    x = x_ref[...]
    if relu_in:
        x = jnp.maximum(x, 0.0)
    o_ref[...] = jnp.dot(x, w_ref[...], preferred_element_type=jnp.float32)


def _matmul(x, w, relu_in=False, bm=1000):
    n, k = x.shape
    f = w.shape[1]
    return pl.pallas_call(
        functools.partial(_mm_body, relu_in=relu_in),
        grid=(n // bm,),
        in_specs=[
            pl.BlockSpec((bm, k), lambda i: (i, 0)),
            pl.BlockSpec((k, f), lambda i: (0, 0)),
        ],
        out_specs=pl.BlockSpec((bm, f), lambda i: (i, 0)),
        out_shape=jax.ShapeDtypeStruct((n, f), jnp.float32),
    )(x, w)


def _norm_body(x_ref, o_ref):
    y = jnp.maximum(x_ref[...], 0.0)
    nrm = jnp.sqrt(jnp.sum(y * y, axis=1, keepdims=True))
    o_ref[...] = y / jnp.maximum(nrm, 1e-12)


def _relu_l2norm(x, bm=1000):
    n, f = x.shape
    return pl.pallas_call(
        _norm_body,
        grid=(n // bm,),
        in_specs=[pl.BlockSpec((bm, f), lambda i: (i, 0))],
        out_specs=pl.BlockSpec((bm, f), lambda i: (i, 0)),
        out_shape=jax.ShapeDtypeStruct((n, f), jnp.float32),
    )(x)


def _relu_body(x_ref, o_ref):
    o_ref[...] = jnp.maximum(x_ref[...], 0.0)


def _relu(x, bm=1000):
    n, f = x.shape
    return pl.pallas_call(
        _relu_body,
        grid=(n // bm,),
        in_specs=[pl.BlockSpec((bm, f), lambda i: (i, 0))],
        out_specs=pl.BlockSpec((bm, f), lambda i: (i, 0)),
        out_shape=jax.ShapeDtypeStruct((n, f), jnp.float32),
    )(x)


def _sc_spmm(sup_sr, sup_tg, e_sr, e_tg, feat):
    mesh = plsc.VectorSubcoreMesh(core_axis_name="c", subcore_axis_name="s")
    nvec = feat // 16
    b_edges, n_chunks, stage_rows = _CFG[feat]
    n_stages = n_chunks // stage_rows

    @functools.partial(
        pl.kernel,
        mesh=mesh,
        out_type=(
            jax.ShapeDtypeStruct((N_PAD, feat), jnp.float32),
            jax.ShapeDtypeStruct((N_PAD, feat), jnp.float32),
        ),
        scratch_types=[
            pltpu.VMEM((stage_rows, 3, b_edges), jnp.int32),
            pltpu.VMEM((b_edges, feat), jnp.float32),
            pltpu.VMEM((b_edges, feat), jnp.float32),
            pltpu.VMEM_SHARED((N_PAD, feat), jnp.float32),
            pltpu.SemaphoreType.DMA,
            pltpu.SemaphoreType.DMA,
            pltpu.SemaphoreType.DMA,
            pltpu.SemaphoreType.DMA,
        ],
        compiler_params=pltpu.CompilerParams(use_tc_tiling_on_sc=False),
    )
    def spmm(sup_sr_h, sup_tg_h, esr_h, etg_h, out_sr_h, out_tg_h,
             idx_v, rows_a, rows_b, acc, gsem_a, gsem_b, ssem_a, ssem_b):
        g = lax.axis_index("c")
        sid = lax.axis_index("s")
        bufs = (rows_a, rows_b)
        gsems = (gsem_a, gsem_b)
        ssems = (ssem_a, ssem_b)

        zr = min(b_edges, ROWS_PER_TILE)
        def zrow(r, c):
            for j in range(nvec):
                rows_a[r, pl.ds(j * 16, 16)] = jnp.zeros((16,), jnp.float32)
            return c
        lax.fori_loop(0, zr, zrow, 0)
        for i in range(ROWS_PER_TILE // zr):
            pltpu.sync_copy(
                rows_a.at[pl.ds(0, zr)],
                acc.at[pl.ds(sid * ROWS_PER_TILE + i * zr, zr)])
        plsc.subcore_barrier()

        def run(sup_h, e_h, out_h):
            def stage(si, c):
                sl = pl.ds(si * stage_rows, stage_rows)
                pltpu.sync_copy(e_h.at[sid, sl], idx_v)

                gd = {0: pltpu.async_copy(sup_h.at[idx_v.at[0, 0]], bufs[0],
                                          gsems[0])}
                sd = {}
                for r in range(stage_rows):
                    b = r % 2
                    gd[r].wait()
                    if r + 1 < stage_rows:
                        nb = (r + 1) % 2
                        if r >= 1:
                            sd[r - 1].wait()
                        gd[r + 1] = pltpu.async_copy(
                            sup_h.at[idx_v.at[r + 1, 0]], bufs[nb], gsems[nb])

                    def edge_group(gi, c3, _r=r, _b=b):
                        wbits = idx_v[_r, 2, pl.ds(gi * 16, 16)]
                        wvec = jax.lax.bitcast_convert_type(wbits, jnp.float32)
                        for i in range(16):
                            wv = wvec[i]
                            e = gi * 16 + i
                            for j in range(nvec):
                                fl = pl.ds(j * 16, 16)
                                bufs[_b][e, fl] = bufs[_b][e, fl] * wv
                        return c3
                    lax.fori_loop(0, b_edges // 16, edge_group, 0)
                    sd[r] = pltpu.async_copy(bufs[b], acc.at[idx_v.at[r, 1]],
                                             ssems[b], add=True)
                if stage_rows >= 2:
                    sd[stage_rows - 2].wait()
                sd[stage_rows - 1].wait()
                return c
            lax.fori_loop(0, n_stages, stage, 0)
            plsc.subcore_barrier()
            pltpu.sync_copy(
                acc.at[pl.ds(sid * ROWS_PER_TILE, ROWS_PER_TILE)],
                out_h.at[pl.ds(sid * ROWS_PER_TILE, ROWS_PER_TILE)])

        @pl.when(g == 0)
        def _():
            run(sup_sr_h, esr_h, out_sr_h)

        @pl.when(g == 1)
        def _():
            run(sup_tg_h, etg_h, out_tg_h)

    return spmm(sup_sr, sup_tg, e_sr, e_tg)


def _prep(ei, ew, feat):
    b_edges, n_chunks, _ = _CFG[feat]
    ept_pad = n_chunks * b_edges

    def shape_one(a):
        a = a.reshape(N_TILES, EPT)
        if ept_pad > EPT:
            a = jnp.pad(a, ((0, 0), (0, ept_pad - EPT)))
        return a.reshape(N_TILES, n_chunks, b_edges)

    src = shape_one(jnp.asarray(ei[0], jnp.int32))
    dst = shape_one(jnp.asarray(ei[1], jnp.int32))
    w = shape_one(jax.lax.bitcast_convert_type(ew.astype(jnp.float32),
                                               jnp.int32))
    return jnp.stack([src, dst, w], axis=2)


def kernel(edge_index_sr, edge_index_tg, edge_weight_sr, edge_weight_tg,
           attr_weight_sr, attr_weight_tg, emb_sr, emb_tg,
           W_s0, W_s1, W_a11, W_a12, W_a2):
    esr128 = _prep(edge_index_sr, edge_weight_sr, 128)
    etg128 = _prep(edge_index_tg, edge_weight_tg, 128)
    esr64 = _prep(edge_index_sr, edge_weight_sr, 64)
    etg64 = _prep(edge_index_tg, edge_weight_tg, 64)

    s_sr = _matmul(emb_sr, W_s0)
    s_tg = _matmul(emb_tg, W_s0)
    a1_sr, a1_tg = _sc_spmm(s_sr, s_tg, esr128, etg128, 128)
    s2_sr = _matmul(a1_sr, W_s1, relu_in=True, bm=1024)
    s2_tg = _matmul(a1_tg, W_s1, relu_in=True, bm=1024)
    a2_sr, a2_tg = _sc_spmm(s2_sr, s2_tg, esr128, etg128, 128)
    sr_s = _relu_l2norm(a2_sr[:N_NODES])
    tg_s = _relu_l2norm(a2_tg[:N_NODES])

    t_sr = _matmul(attr_weight_sr, W_a11)
    t_tg = _matmul(attr_weight_tg, W_a12)
    b1_sr, b1_tg = _sc_spmm(t_sr, t_tg, esr64, etg64, 64)
    t2_sr = _matmul(b1_sr, W_a2, relu_in=True, bm=1024)
    t2_tg = _matmul(b1_tg, W_a2, relu_in=True, bm=1024)
    b2_sr, b2_tg = _sc_spmm(t2_sr, t2_tg, esr64, etg64, 64)
    sr_a = _relu(b2_sr[:N_NODES])
    tg_a = _relu(b2_tg[:N_NODES])

    return (sr_s, tg_s, sr_a, tg_a)
